# baseline (speedup 1.0000x reference)
import functools

import jax
import jax.numpy as jnp
from jax import lax
from jax.experimental import pallas as pl
from jax.experimental.pallas import tpu as pltpu
from jax.experimental.pallas import tpu_sc as plsc

B = 16
H = 12
S = 576
D = 768
K = 128


def _score_topk_body(sl_ref, idx_ref):
    acc = sl_ref[:, 0, :]
    for h in range(1, H):
        acc = acc + sl_ref[:, h, :]
    v = acc / jnp.float32(H)

    iota_i = lax.broadcasted_iota(jnp.int32, (B, 1, S), 2)
    vi = v[:, None, :]
    rank = jnp.zeros((B, S), jnp.float32)
    JC = 96
    for j0 in range(0, S, JC):
        vj = v[:, j0 : j0 + JC][:, :, None]
        jidx = lax.broadcasted_iota(jnp.int32, (B, JC, 1), 1) + j0
        before = (vj > vi) | ((vj == vi) & (jidx < iota_i))
        rank = rank + jnp.sum(jnp.where(before, 1.0, 0.0), axis=1)
    ranki = rank.astype(jnp.int32)

    kiota = lax.broadcasted_iota(jnp.int32, (B, 1, K), 2)
    out = jnp.zeros((B, K), jnp.int32)
    IC = 192
    for i0 in range(0, S, IC):
        rc = ranki[:, i0 : i0 + IC][:, :, None]
        ii = lax.broadcasted_iota(jnp.int32, (B, IC, 1), 1) + i0
        out = out + jnp.sum(jnp.where(rc == kiota, ii, 0), axis=1)

    row_k = lax.broadcasted_iota(jnp.int32, (B, K), 0)
    idx_ref[...] = out + S * row_k


_NC = 2
_NS = 16
_NW = _NC * _NS
ROWS = B * K
RPW = ROWS // _NW


def _sc_gather_body(table_hbm, idx_hbm, out_hbm, idx_v, rows_v, sem):
    wid = lax.axis_index("s") * _NC + lax.axis_index("c")
    base = wid * RPW
    pltpu.sync_copy(idx_hbm.at[pl.ds(base, RPW)], idx_v)
    pltpu.async_copy(table_hbm.at[idx_v], rows_v, sem).wait()
    pltpu.sync_copy(rows_v, out_hbm.at[pl.ds(base, RPW)])


@functools.cache
def _sc_gather():
    return pl.kernel(
        _sc_gather_body,
        mesh=plsc.VectorSubcoreMesh(core_axis_name="c", subcore_axis_name="s"),
        out_type=jax.ShapeDtypeStruct((ROWS, D), jnp.float32),
        scratch_types=[
            pltpu.VMEM((RPW,), jnp.int32),
            pltpu.VMEM((RPW, D), jnp.float32),
            pltpu.SemaphoreType.DMA,
        ],
    )


def kernel(feats, attns):
    sl = attns[:, :, 0, 1:]
    idx = pl.pallas_call(
        _score_topk_body,
        in_specs=[pl.BlockSpec((B, H, S), lambda: (0, 0, 0))],
        out_specs=pl.BlockSpec((B, K), lambda: (0, 0)),
        out_shape=jax.ShapeDtypeStruct((B, K), jnp.int32),
    )(sl)
    flat_idx = idx.reshape(ROWS)
    table = feats.reshape(B * S, D)
    out = _sc_gather()(table, flat_idx)
    return out.reshape(B, K, D)


# speedup vs baseline: 1.0220x; 1.0220x over previous
import functools

import jax
import jax.numpy as jnp
from jax import lax
from jax.experimental import pallas as pl
from jax.experimental.pallas import tpu as pltpu
from jax.experimental.pallas import tpu_sc as plsc

B = 16
H = 12
S = 576
D = 768
K = 128


def _score_topk_body(sl_ref, idx_ref):
    acc = sl_ref[:, 0, :]
    for h in range(1, H):
        acc = acc + sl_ref[:, h, :]
    v = acc / jnp.float32(H)

    iota_i = lax.broadcasted_iota(jnp.int32, (B, 1, S), 2)
    vi = v[:, None, :]
    rank = jnp.zeros((B, S), jnp.float32)
    JC = 576
    for j0 in range(0, S, JC):
        vj = v[:, j0 : j0 + JC][:, :, None]
        jidx = lax.broadcasted_iota(jnp.int32, (B, JC, 1), 1) + j0
        before = (vj > vi) | ((vj == vi) & (jidx < iota_i))
        rank = rank + jnp.sum(jnp.where(before, 1.0, 0.0), axis=1)
    ranki = rank.astype(jnp.int32)

    kiota = lax.broadcasted_iota(jnp.int32, (B, 1, K), 2)
    out = jnp.zeros((B, K), jnp.int32)
    IC = 288
    for i0 in range(0, S, IC):
        rc = ranki[:, i0 : i0 + IC][:, :, None]
        ii = lax.broadcasted_iota(jnp.int32, (B, IC, 1), 1) + i0
        out = out + jnp.sum(jnp.where(rc == kiota, ii, 0), axis=1)

    row_k = lax.broadcasted_iota(jnp.int32, (B, K), 0)
    idx_ref[...] = out + S * row_k


_NC = 2
_NS = 16
_NW = _NC * _NS
ROWS = B * K
RPW = ROWS // _NW


def _sc_gather_body(table_hbm, idx_hbm, out_hbm, idx_v, rows_v, sem):
    wid = lax.axis_index("s") * _NC + lax.axis_index("c")
    base = wid * RPW
    pltpu.sync_copy(idx_hbm.at[pl.ds(base, RPW)], idx_v)
    pltpu.async_copy(table_hbm.at[idx_v], rows_v, sem).wait()
    pltpu.sync_copy(rows_v, out_hbm.at[pl.ds(base, RPW)])


@functools.cache
def _sc_gather():
    return pl.kernel(
        _sc_gather_body,
        mesh=plsc.VectorSubcoreMesh(core_axis_name="c", subcore_axis_name="s"),
        out_type=jax.ShapeDtypeStruct((ROWS, D), jnp.float32),
        scratch_types=[
            pltpu.VMEM((RPW,), jnp.int32),
            pltpu.VMEM((RPW, D), jnp.float32),
            pltpu.SemaphoreType.DMA,
        ],
    )


def kernel(feats, attns):
    sl = attns[:, :, 0, 1:]
    idx = pl.pallas_call(
        _score_topk_body,
        in_specs=[pl.BlockSpec((B, H, S), lambda: (0, 0, 0))],
        out_specs=pl.BlockSpec((B, K), lambda: (0, 0)),
        out_shape=jax.ShapeDtypeStruct((B, K), jnp.int32),
    )(sl)
    flat_idx = idx.reshape(ROWS)
    table = feats.reshape(B * S, D)
    out = _sc_gather()(table, flat_idx)
    return out.reshape(B, K, D)
